# baseline (speedup 1.0000x reference)
import functools

import jax
import jax.numpy as jnp
from jax import lax
from jax.experimental import pallas as pl
from jax.experimental.pallas import tpu as pltpu
from jax.experimental.pallas import tpu_sc as plsc

N = 10000
E = 160000
NS = 16
NC = 2
EDGES_PER_SUB = E // NS
RC = 80
EC = 50
DRA = 5
RCH = EDGES_PER_SUB // (EC * DRA)
BR = 1000
GI = N // BR

_mesh = plsc.VectorSubcoreMesh(core_axis_name="c", subcore_axis_name="s")


def _sc_aggregate(g, src2, dst):
    srcr = src2.reshape(NC * NS * DRA, RCH, EC)
    dstr = dst.reshape(NS * DRA, RCH, EC)

    @functools.partial(
        pl.kernel,
        out_type=jax.ShapeDtypeStruct((2 * N, 128), jnp.float32),
        mesh=_mesh,
        scratch_types=[
            pltpu.VMEM((RCH, EC), jnp.int32),
            pltpu.VMEM((RCH, EC), jnp.int32),
            pltpu.VMEM((EC, 128), jnp.float32),
            pltpu.VMEM((EC, 128), jnp.float32),
            pltpu.VMEM((EC, 128), jnp.float32),
            pltpu.VMEM((EC, 128), jnp.float32),
            pltpu.VMEM_SHARED((N, 128), jnp.float32),
            pltpu.SemaphoreType.DMA,
            pltpu.SemaphoreType.DMA,
            pltpu.SemaphoreType.DMA,
            pltpu.SemaphoreType.DMA,
            pltpu.SemaphoreType.DMA,
            pltpu.SemaphoreType.DMA,
            pltpu.SemaphoreType.DMA,
            pltpu.SemaphoreType.DMA,
        ],
    )
    def k(g_hbm, src_hbm, dst_hbm, out_hbm, sidx, didx, r0_, r1_, r2_, r3_,
          acc_sh, g0, g1, g2, g3, s0, s1, s2, s3):
        cid = lax.axis_index("c")
        sid = lax.axis_index("s")
        bufs = ((r0_, g0, s0), (r1_, g1, s1), (r2_, g2, s2), (r3_, g3, s3))

        @pl.loop(sid * RC, N, step=NS * RC)
        def _(r0):
            pltpu.sync_copy(g_hbm.at[pl.ds(cid * N + r0, RC)],
                            acc_sh.at[pl.ds(r0, RC)])

        plsc.subcore_barrier()

        def gstart(j, rows, sem):
            pltpu.async_copy(g_hbm.at[sidx.at[j]], rows, sem)

        def gwait(j, rows, sem):
            pltpu.make_async_copy(g_hbm.at[sidx.at[j]], rows, sem).wait()

        def sstart(j, rows, sem):
            pltpu.async_copy(rows, acc_sh.at[didx.at[j]], sem, add=True)

        def swait(j, rows, sem):
            pltpu.make_async_copy(rows, acc_sh.at[didx.at[j]], sem).wait()

        @pl.loop(0, DRA)
        def _(r):
            pltpu.sync_copy(src_hbm.at[(cid * NS + sid) * DRA + r], sidx)
            pltpu.sync_copy(dst_hbm.at[sid * DRA + r], didx)
            for k in range(4):
                gstart(k, bufs[k][0], bufs[k][1])

            @pl.loop(0, RCH - 4, step=4)
            def _(j):
                for k in range(4):
                    gwait(j + k, bufs[k][0], bufs[k][1])
                    sstart(j + k, bufs[k][0], bufs[k][2])
                for k in range(4):
                    swait(j + k, bufs[k][0], bufs[k][2])
                    gstart(j + 4 + k, bufs[k][0], bufs[k][1])

            jt = RCH - 4
            for k in range(4):
                gwait(jt + k, bufs[k][0], bufs[k][1])
                sstart(jt + k, bufs[k][0], bufs[k][2])
            for k in range(4):
                swait(jt + k, bufs[k][0], bufs[k][2])

        plsc.subcore_barrier()

        @pl.loop(sid * RC, N, step=NS * RC)
        def _(r0):
            pltpu.sync_copy(acc_sh.at[pl.ds(r0, RC)],
                            out_hbm.at[pl.ds(cid * N + r0, RC)])

    return k(g, srcr, dstr)


def _tc1_body(x_ref, w1_ref, p_ref, g_ref):
    dinv = lax.rsqrt(p_ref[:, 0:1])
    h = lax.dot_general(x_ref[...], w1_ref[...], (((1,), (0,)), ((), ())),
                        preferred_element_type=jnp.float32)
    g_ref[...] = h * dinv


def _tc1(x, W1, degp):
    return pl.pallas_call(
        _tc1_body,
        grid=(GI, 2),
        in_specs=[
            pl.BlockSpec((BR, 512), lambda i, c: (i, 0)),
            pl.BlockSpec((512, 128), lambda i, c: (0, c)),
            pl.BlockSpec((BR, 128), lambda i, c: (i, 0)),
        ],
        out_specs=pl.BlockSpec((BR, 128), lambda i, c: (c * GI + i, 0)),
        out_shape=jax.ShapeDtypeStruct((2 * N, 128), jnp.float32),
    )(x, W1, degp)


def _tc2_body(alo_ref, ahi_ref, p_ref, w2a_ref, w2b_ref, blo_ref, bhi_ref,
              g2_ref):
    dinv = lax.rsqrt(p_ref[:, 0:1])
    a0 = jnp.maximum(alo_ref[...] * dinv + blo_ref[0], 0.0)
    a1 = jnp.maximum(ahi_ref[...] * dinv + bhi_ref[0], 0.0)
    h2 = (lax.dot_general(a0, w2a_ref[...], (((1,), (0,)), ((), ())),
                          preferred_element_type=jnp.float32)
          + lax.dot_general(a1, w2b_ref[...], (((1,), (0,)), ((), ())),
                            preferred_element_type=jnp.float32))
    g2_ref[...] = h2 * dinv


def _tc2(acc1, degp, b1, W2):
    b1r = b1.reshape(2, 1, 128)
    return pl.pallas_call(
        _tc2_body,
        grid=(GI, 2),
        in_specs=[
            pl.BlockSpec((BR, 128), lambda i, c: (i, 0)),
            pl.BlockSpec((BR, 128), lambda i, c: (i + GI, 0)),
            pl.BlockSpec((BR, 128), lambda i, c: (i, 0)),
            pl.BlockSpec((128, 128), lambda i, c: (0, c)),
            pl.BlockSpec((128, 128), lambda i, c: (1, c)),
            pl.BlockSpec((1, 1, 128), lambda i, c: (0, 0, 0)),
            pl.BlockSpec((1, 1, 128), lambda i, c: (1, 0, 0)),
        ],
        out_specs=pl.BlockSpec((BR, 128), lambda i, c: (c * GI + i, 0)),
        out_shape=jax.ShapeDtypeStruct((2 * N, 128), jnp.float32),
    )(acc1, acc1, degp, W2, W2, b1r, b1r)


def _tc3_body(acc_ref, p_ref, b_ref, out_ref):
    dinv = lax.rsqrt(p_ref[:, 0:1])
    out_ref[...] = jnp.maximum(acc_ref[...] * dinv + b_ref[0], 0.0)


def _tc3(acc2, degp, b2):
    b2r = b2.reshape(2, 1, 128)
    return pl.pallas_call(
        _tc3_body,
        grid=(GI, 2),
        in_specs=[
            pl.BlockSpec((BR, 128), lambda i, c: (c * GI + i, 0)),
            pl.BlockSpec((BR, 128), lambda i, c: (i, 0)),
            pl.BlockSpec((1, 1, 128), lambda i, c: (c, 0, 0)),
        ],
        out_specs=pl.BlockSpec((BR, 128), lambda i, c: (i, c)),
        out_shape=jax.ShapeDtypeStruct((N, 256), jnp.float32),
    )(acc2, degp, b2r)


def kernel(x, edge_index, W1, b1, W2, b2):
    src = edge_index[0]
    dst = edge_index[1]
    src2 = jnp.concatenate([src, src + N])

    degp = _sc_aggregate(jnp.ones((2 * N, 128), jnp.float32), src2, dst)
    g1 = _tc1(x, W1, degp)
    acc1 = _sc_aggregate(g1, src2, dst)
    g2 = _tc2(acc1, degp, b1, W2)
    acc2 = _sc_aggregate(g2, src2, dst)
    return _tc3(acc2, degp, b2)
